# Initial kernel scaffold; baseline (speedup 1.0000x reference)
import jax
import jax.numpy as jnp
from jax.experimental import pallas as pl


def kernel(input_batch, input_lengths, embedding_table):
    raise NotImplementedError("write your pallas kernel here")


# speedup vs baseline: 1.1469x; 1.1469x over previous
import functools

import jax
import jax.numpy as jnp
from jax import lax
from jax.experimental import pallas as pl
from jax.experimental.pallas import tpu as pltpu, tpu_sc as plsc

_VOCAB = 1000000
_D = 32
_B = 4096
_T = 200

_NC = 2
_NS = 16
_NW = _NC * _NS
_RW = _B // _NW
_G = 100
_GPR = _T // _G
_RB = 8
_NG = _RB * _GPR
_NCHUNK = _RW // _RB
_NGROUPS = _B * _GPR


def _body(table_hbm, idx_hbm, emb_hbm, avg_hbm, idx_v, rows_v, avg_v, sem):
    wid = lax.axis_index("s") * _NC + lax.axis_index("c")
    base_g = wid * (_RW * _GPR)

    def chunk_body(c, carry):
        g0 = base_g + c * _NG
        pltpu.sync_copy(idx_hbm.at[pl.ds(g0, _NG)], idx_v)
        descs = [
            pltpu.async_copy(table_hbm.at[idx_v.at[j]], rows_v.at[j], sem)
            for j in range(_NG)
        ]
        for d in descs:
            d.wait()
        for r in range(_RB):
            def kbody(k, accs):
                a0, a1 = accs
                for u in range(4):
                    kk = k * 4 + u
                    a0 = (a0 + rows_v[2 * r, kk, pl.ds(0, 16)]
                          + rows_v[2 * r + 1, kk, pl.ds(0, 16)])
                    a1 = (a1 + rows_v[2 * r, kk, pl.ds(16, 16)]
                          + rows_v[2 * r + 1, kk, pl.ds(16, 16)])
                return (a0, a1)
            zero = jnp.zeros((16,), jnp.float32)
            a0, a1 = lax.fori_loop(0, _G // 4, kbody, (zero, zero))
            row = c * _RB + r
            avg_v[row, pl.ds(0, 16)] = a0 * (1.0 / _T)
            avg_v[row, pl.ds(16, 16)] = a1 * (1.0 / _T)
        pltpu.sync_copy(rows_v, emb_hbm.at[pl.ds(g0, _NG)])
        return carry

    lax.fori_loop(0, _NCHUNK, chunk_body, 0)
    pltpu.sync_copy(avg_v, avg_hbm.at[pl.ds(wid * _RW, _RW)])


_sc_call = functools.partial(
    pl.kernel,
    out_type=(
        jax.ShapeDtypeStruct((_NGROUPS, _G, _D), jnp.float32),
        jax.ShapeDtypeStruct((_B, _D), jnp.float32),
    ),
    mesh=plsc.VectorSubcoreMesh(
        core_axis_name="c", subcore_axis_name="s",
        num_cores=_NC, num_subcores=_NS),
    scratch_types=[
        pltpu.VMEM((_NG, _G), jnp.int32),
        pltpu.VMEM((_NG, _G, _D), jnp.float32),
        pltpu.VMEM((_RW, _D), jnp.float32),
        pltpu.SemaphoreType.DMA,
    ],
    compiler_params=pltpu.CompilerParams(use_tc_tiling_on_sc=False),
)(_body)


@jax.jit
def kernel(input_batch, input_lengths, embedding_table):
    idx2 = input_batch.reshape(_NGROUPS, _G)
    emb, avg = _sc_call(embedding_table, idx2)
    return (avg, emb.reshape(_B, _T, _D), input_lengths)
